# baseline (speedup 1.0000x reference)
import jax
import jax.numpy as jnp
from jax import lax
from jax.experimental import pallas as pl
from jax.experimental.pallas import tpu as pltpu, tpu_sc as plsc

N = 100000
M = 9
NC = 2
NS = 16
NW = NC * NS
BLK = 160
NBLK = 20
IDXB = BLK * M
CPT = BLK * NBLK
NP = NW * CPT
BCH = 800
B0 = 20
B1 = 2 * NBLK - B0

_params = pltpu.CompilerParams(
    needs_layout_passes=False, use_tc_tiling_on_sc=False
)


def _make_mesh():
    return plsc.VectorSubcoreMesh(
        core_axis_name="c", subcore_axis_name="s", num_cores=NC, num_subcores=NS
    )


def _wid():
    return lax.axis_index("s") * NC + lax.axis_index("c")


def _full16(v):
    return jnp.full((16,), v, jnp.int32)


def _phase1(xt, utf, u1f, invt, nidx, upf, usf, out_t2, out_part,
            idx_a, idx_b, buf_a, buf_b, own_a, own_b, up_a, up_b, us_a, us_b,
            outb, resv, tabs, bx, bu, b1v, bi, bc,
            sb, sia, sib, sga, sgb, soa, sob, spa, spb, sqa, sqb):
    w = _wid()
    c = lax.axis_index("c")
    sidx = lax.axis_index("s")
    myblk = jnp.where(c == 0, B0, B1)
    gbase = jnp.where(c == 0, sidx * B0, NS * B0 + sidx * B1)
    iota = lax.iota(jnp.int32, 16)

    def bchunk(ch, carry):
        rbase = sidx * (NP // NS) + ch * BCH
        pltpu.async_copy(xt.at[:, pl.ds(rbase, BCH)], bx, sb)
        pltpu.async_copy(utf.at[pl.ds(rbase, BCH)], bu, sb)
        pltpu.async_copy(u1f.at[pl.ds(rbase, BCH)], b1v, sb)
        pltpu.async_copy(invt.at[:, pl.ds(rbase, BCH)], bi, sb)
        pltpu.make_async_copy(xt.at[:, pl.ds(0, BCH)], bx, sb).wait()
        pltpu.make_async_copy(utf.at[pl.ds(0, BCH)], bu, sb).wait()
        pltpu.make_async_copy(u1f.at[pl.ds(0, BCH)], b1v, sb).wait()
        pltpu.make_async_copy(invt.at[:, pl.ds(0, BCH)], bi, sb).wait()
        for p in range(0, BCH, 16):
            rows = iota + p
            plsc.store_scatter(bc, [rows, _full16(0)], bx[0, pl.ds(p, 16)])
            plsc.store_scatter(bc, [rows, _full16(1)], bx[1, pl.ds(p, 16)])
            plsc.store_scatter(bc, [rows, _full16(2)], bu[pl.ds(p, 16)])
            plsc.store_scatter(bc, [rows, _full16(3)], b1v[pl.ds(p, 16)])
            plsc.store_scatter(bc, [rows, _full16(4)], bi[0, pl.ds(p, 16)])
            plsc.store_scatter(bc, [rows, _full16(5)], bi[1, pl.ds(p, 16)])
            plsc.store_scatter(bc, [rows, _full16(6)], bi[2, pl.ds(p, 16)])
            plsc.store_scatter(bc, [rows, _full16(7)], bi[3, pl.ds(p, 16)])
        pltpu.sync_copy(bc, tabs.at[pl.ds(rbase, BCH)])
        return carry

    lax.fori_loop(0, NP // NS // BCH, bchunk, 0)
    plsc.subcore_barrier()

    sets = [(idx_a, buf_a, own_a, up_a, us_a, sia, sga, soa, spa, sqa),
            (idx_b, buf_b, own_b, up_b, us_b, sib, sgb, sob, spb, sqb)]

    def issue_idx(b, s):
        idx_v, buf, ownv, upv, usv, si, sg, so, sp, sq = s
        base = (gbase + b) * BLK
        for m in range(M):
            pltpu.async_copy(nidx.at[m, pl.ds(base, BLK)],
                             idx_v.at[pl.ds(m * BLK, BLK)], si)

    def wait_idx_fire(b, s):
        idx_v, buf, ownv, upv, usv, si, sg, so, sp, sq = s
        base = (gbase + b) * BLK
        for m in range(M):
            pltpu.make_async_copy(nidx.at[m, pl.ds(0, BLK)],
                                  idx_v.at[pl.ds(m * BLK, BLK)], si).wait()
        pltpu.async_copy(tabs.at[idx_v], buf, sg)
        pltpu.async_copy(tabs.at[pl.ds(base, BLK)], ownv, so)
        pltpu.async_copy(upf.at[pl.ds(base, BLK)], upv, sp)
        pltpu.async_copy(usf.at[pl.ds(base, BLK)], usv, sq)

    def wait(s):
        idx_v, buf, ownv, upv, usv, si, sg, so, sp, sq = s
        pltpu.make_async_copy(tabs.at[idx_v], buf, sg).wait()
        pltpu.make_async_copy(tabs.at[pl.ds(0, BLK)], ownv, so).wait()
        pltpu.make_async_copy(upf.at[pl.ds(0, BLK)], upv, sp).wait()
        pltpu.make_async_copy(usf.at[pl.ds(0, BLK)], usv, sq).wait()

    def compute(b, s, accu):
        idx_v, buf, ownv, upv, usv, si, sg, so, sp, sq = s
        base = (gbase + b) * BLK
        for p in range(0, BLK, 16):
            rows = iota + p
            xi0 = plsc.load_gather(ownv, [rows, _full16(0)])
            xi1 = plsc.load_gather(ownv, [rows, _full16(1)])
            uti = plsc.load_gather(ownv, [rows, _full16(2)])
            u1i = plsc.load_gather(ownv, [rows, _full16(3)])
            s00 = jnp.zeros((16,), jnp.float32)
            s01 = jnp.zeros((16,), jnp.float32)
            s10 = jnp.zeros((16,), jnp.float32)
            s11 = jnp.zeros((16,), jnp.float32)
            for m in range(M):
                rm = rows + m * BLK
                xj0 = plsc.load_gather(buf, [rm, _full16(0)])
                xj1 = plsc.load_gather(buf, [rm, _full16(1)])
                uj = plsc.load_gather(buf, [rm, _full16(2)])
                u1j = plsc.load_gather(buf, [rm, _full16(3)])
                xd0 = xj0 - xi0
                xd1 = xj1 - xi1
                ud = uj - uti
                ud1 = u1j - u1i
                s00 = s00 + ud * xd0
                s01 = s01 + ud * xd1
                s10 = s10 + ud1 * xd0
                s11 = s11 + ud1 * xd1
            ia = plsc.load_gather(ownv, [rows, _full16(4)])
            ib = plsc.load_gather(ownv, [rows, _full16(5)])
            ic = plsc.load_gather(ownv, [rows, _full16(6)])
            id_ = plsc.load_gather(ownv, [rows, _full16(7)])
            gx = s00 * ia + s01 * ic
            gy = s00 * ib + s01 * id_
            gx1 = s10 * ia + s11 * ic
            gy1 = s10 * ib + s11 * id_
            plsc.store_scatter(outb, [rows, _full16(0)], xi0)
            plsc.store_scatter(outb, [rows, _full16(1)], xi1)
            plsc.store_scatter(outb, [rows, _full16(2)], gx)
            plsc.store_scatter(outb, [rows, _full16(3)], gy)
            plsc.store_scatter(outb, [rows, _full16(4)], gx1)
            plsc.store_scatter(outb, [rows, _full16(5)], gy1)
            plsc.store_scatter(outb, [rows, _full16(6)], uti)
            plsc.store_scatter(outb, [rows, _full16(7)], u1i)
            du = upv[pl.ds(p, 16)] - usv[pl.ds(p, 16)]
            accu = accu + du * du
        pltpu.sync_copy(outb, out_t2.at[pl.ds(base, BLK)])
        return accu

    @pl.when(myblk >= 2)
    def _():
        issue_idx(0, sets[0])
        issue_idx(1, sets[1])
        wait_idx_fire(0, sets[0])

    zero = jnp.zeros((16,), jnp.float32)

    def body(g, accu):
        b0 = 2 * g
        for sub in (0, 1):
            b = b0 + sub
            sq_, snx = sets[sub], sets[1 - sub]

            @pl.when(b + 1 < myblk)
            def _():
                wait_idx_fire(b + 1, snx)

            wait(sq_)

            @pl.when(b + 2 < myblk)
            def _():
                issue_idx(b + 2, sq_)

            accu = compute(b, sq_, accu)
        return accu

    accu = lax.fori_loop(0, myblk // 2, body, zero)
    resv[...] = accu
    pltpu.sync_copy(resv, out_part.at[w])


def _phase2(table2, invt, nidx, out_part,
            idx_a, idx_b, buf_a, buf_b, own_a, own_b, iv_a, iv_b, resv, tabs,
            sia, sib, sga, sgb, soa, sob, spa, spb):
    w = _wid()
    c = lax.axis_index("c")
    sidx = lax.axis_index("s")
    myblk = jnp.where(c == 0, B0, B1)
    gbase = jnp.where(c == 0, sidx * B0, NS * B0 + sidx * B1)
    iota = lax.iota(jnp.int32, 16)
    slc = NP // NS
    pltpu.sync_copy(table2.at[pl.ds(sidx * slc, slc)],
                    tabs.at[pl.ds(sidx * slc, slc)])
    plsc.subcore_barrier()

    sets = [(idx_a, buf_a, own_a, iv_a, sia, sga, soa, spa),
            (idx_b, buf_b, own_b, iv_b, sib, sgb, sob, spb)]

    def issue_idx(b, s):
        idx_v, buf, ownv, ivv, si, sg, so, sp = s
        base = (gbase + b) * BLK
        for m in range(M):
            pltpu.async_copy(nidx.at[m, pl.ds(base, BLK)],
                             idx_v.at[pl.ds(m * BLK, BLK)], si)

    def wait_idx_fire(b, s):
        idx_v, buf, ownv, ivv, si, sg, so, sp = s
        base = (gbase + b) * BLK
        for m in range(M):
            pltpu.make_async_copy(nidx.at[m, pl.ds(0, BLK)],
                                  idx_v.at[pl.ds(m * BLK, BLK)], si).wait()
        pltpu.async_copy(tabs.at[idx_v], buf, sg)
        pltpu.async_copy(tabs.at[pl.ds(base, BLK)], ownv, so)
        pltpu.async_copy(invt.at[:, pl.ds(base, BLK)], ivv, sp)

    def wait(s):
        idx_v, buf, ownv, ivv, si, sg, so, sp = s
        pltpu.make_async_copy(tabs.at[idx_v], buf, sg).wait()
        pltpu.make_async_copy(tabs.at[pl.ds(0, BLK)], ownv, so).wait()
        pltpu.make_async_copy(invt.at[:, pl.ds(0, BLK)], ivv, sp).wait()

    def compute(b, s, accf):
        idx_v, buf, ownv, ivv, si, sg, so, sp = s
        base = (gbase + b) * BLK
        for p in range(0, BLK, 16):
            rows = iota + p
            xi0 = plsc.load_gather(ownv, [rows, _full16(0)])
            xi1 = plsc.load_gather(ownv, [rows, _full16(1)])
            gi0 = plsc.load_gather(ownv, [rows, _full16(2)])
            gi1 = plsc.load_gather(ownv, [rows, _full16(3)])
            gi2 = plsc.load_gather(ownv, [rows, _full16(4)])
            gi3 = plsc.load_gather(ownv, [rows, _full16(5)])
            uti = plsc.load_gather(ownv, [rows, _full16(6)])
            u1i = plsc.load_gather(ownv, [rows, _full16(7)])
            a00 = jnp.zeros((16,), jnp.float32)
            a01 = jnp.zeros((16,), jnp.float32)
            a10 = jnp.zeros((16,), jnp.float32)
            a11 = jnp.zeros((16,), jnp.float32)
            b00 = jnp.zeros((16,), jnp.float32)
            b01 = jnp.zeros((16,), jnp.float32)
            b10 = jnp.zeros((16,), jnp.float32)
            b11 = jnp.zeros((16,), jnp.float32)
            for m in range(M):
                rm = rows + m * BLK
                xj0 = plsc.load_gather(buf, [rm, _full16(0)])
                xj1 = plsc.load_gather(buf, [rm, _full16(1)])
                gj0 = plsc.load_gather(buf, [rm, _full16(2)])
                gj1 = plsc.load_gather(buf, [rm, _full16(3)])
                gj2 = plsc.load_gather(buf, [rm, _full16(4)])
                gj3 = plsc.load_gather(buf, [rm, _full16(5)])
                xd0 = xj0 - xi0
                xd1 = xj1 - xi1
                gd0 = gj0 - gi0
                gd1 = gj1 - gi1
                gd2 = gj2 - gi2
                gd3 = gj3 - gi3
                a00 = a00 + gd0 * xd0
                a01 = a01 + gd0 * xd1
                a10 = a10 + gd1 * xd0
                a11 = a11 + gd1 * xd1
                b00 = b00 + gd2 * xd0
                b01 = b01 + gd2 * xd1
                b10 = b10 + gd3 * xd0
                b11 = b11 + gd3 * xd1
            ia = ivv[0, pl.ds(p, 16)]
            ib = ivv[1, pl.ds(p, 16)]
            ic = ivv[2, pl.ds(p, 16)]
            id_ = ivv[3, pl.ds(p, 16)]
            lap = a00 * ia + a01 * ic + a10 * ib + a11 * id_
            lap1 = b00 * ia + b01 * ic + b10 * ib + b11 * id_
            f = u1i - uti - 0.01 * (
                0.01 * lap + uti - uti * uti * uti
                + 0.01 * lap1 + u1i - u1i * u1i * u1i
            )
            gid = base + p + iota
            f = jnp.where(gid < N, f, 0.0)
            accf = accf + f * f
        return accf

    @pl.when(myblk >= 2)
    def _():
        issue_idx(0, sets[0])
        issue_idx(1, sets[1])
        wait_idx_fire(0, sets[0])

    zero = jnp.zeros((16,), jnp.float32)

    def body(g, accf):
        b0 = 2 * g
        for sub in (0, 1):
            b = b0 + sub
            sq_, snx = sets[sub], sets[1 - sub]

            @pl.when(b + 1 < myblk)
            def _():
                wait_idx_fire(b + 1, snx)

            wait(sq_)

            @pl.when(b + 2 < myblk)
            def _():
                issue_idx(b + 2, sq_)

            accf = compute(b, sq_, accf)
        return accf

    accf = lax.fori_loop(0, myblk // 2, body, zero)
    resv[...] = accf
    pltpu.sync_copy(resv, out_part.at[w])


def kernel(up, usol, ut, x_to_train_f, ut1, n_index, inv_mat):
    pad = NP - N
    xt = jnp.pad(x_to_train_f.T, ((0, 0), (0, pad)))
    utf = jnp.pad(ut.reshape(-1), (0, pad))
    u1f = jnp.pad(ut1.reshape(-1), (0, pad))
    invt = jnp.pad(inv_mat.transpose(1, 2, 0).reshape(4, N),
                   ((0, 0), (0, pad)))
    nidx = jnp.pad(n_index.astype(jnp.int32).T, ((0, 0), (0, pad)))
    upf = jnp.pad(up.reshape(-1), (0, pad))
    usf = jnp.pad(usol.reshape(-1), (0, pad))

    p1 = pl.kernel(
        _phase1,
        out_type=(
            jax.ShapeDtypeStruct((NP, 8), jnp.float32),
            jax.ShapeDtypeStruct((NW, 16), jnp.float32),
        ),
        mesh=_make_mesh(),
        compiler_params=_params,
        scratch_types=[
            pltpu.VMEM((IDXB,), jnp.int32),
            pltpu.VMEM((IDXB,), jnp.int32),
            pltpu.VMEM((IDXB, 8), jnp.float32),
            pltpu.VMEM((IDXB, 8), jnp.float32),
            pltpu.VMEM((BLK, 8), jnp.float32),
            pltpu.VMEM((BLK, 8), jnp.float32),
            pltpu.VMEM((BLK,), jnp.float32),
            pltpu.VMEM((BLK,), jnp.float32),
            pltpu.VMEM((BLK,), jnp.float32),
            pltpu.VMEM((BLK,), jnp.float32),
            pltpu.VMEM((BLK, 8), jnp.float32),
            pltpu.VMEM((16,), jnp.float32),
            pltpu.VMEM_SHARED((NP, 8), jnp.float32),
            pltpu.VMEM((2, BCH), jnp.float32),
            pltpu.VMEM((BCH,), jnp.float32),
            pltpu.VMEM((BCH,), jnp.float32),
            pltpu.VMEM((4, BCH), jnp.float32),
            pltpu.VMEM((BCH, 8), jnp.float32),
            pltpu.SemaphoreType.DMA,
            pltpu.SemaphoreType.DMA,
            pltpu.SemaphoreType.DMA,
            pltpu.SemaphoreType.DMA,
            pltpu.SemaphoreType.DMA,
            pltpu.SemaphoreType.DMA,
            pltpu.SemaphoreType.DMA,
            pltpu.SemaphoreType.DMA,
            pltpu.SemaphoreType.DMA,
            pltpu.SemaphoreType.DMA,
            pltpu.SemaphoreType.DMA,
        ],
    )
    table2, part1 = p1(xt, utf, u1f, invt, nidx, upf, usf)

    p2 = pl.kernel(
        _phase2,
        out_type=jax.ShapeDtypeStruct((NW, 16), jnp.float32),
        mesh=_make_mesh(),
        compiler_params=_params,
        scratch_types=[
            pltpu.VMEM((IDXB,), jnp.int32),
            pltpu.VMEM((IDXB,), jnp.int32),
            pltpu.VMEM((IDXB, 8), jnp.float32),
            pltpu.VMEM((IDXB, 8), jnp.float32),
            pltpu.VMEM((BLK, 8), jnp.float32),
            pltpu.VMEM((BLK, 8), jnp.float32),
            pltpu.VMEM((4, BLK), jnp.float32),
            pltpu.VMEM((4, BLK), jnp.float32),
            pltpu.VMEM((16,), jnp.float32),
            pltpu.VMEM_SHARED((NP, 8), jnp.float32),
            pltpu.SemaphoreType.DMA,
            pltpu.SemaphoreType.DMA,
            pltpu.SemaphoreType.DMA,
            pltpu.SemaphoreType.DMA,
            pltpu.SemaphoreType.DMA,
            pltpu.SemaphoreType.DMA,
            pltpu.SemaphoreType.DMA,
            pltpu.SemaphoreType.DMA,
        ],
    )
    part2 = p2(table2, invt, nidx)
    return jnp.sum(part1) + 4.0 * jnp.sum(part2)


# speedup vs baseline: 1.1422x; 1.1422x over previous
import jax
import jax.numpy as jnp
from jax import lax
from jax.experimental import pallas as pl
from jax.experimental.pallas import tpu as pltpu, tpu_sc as plsc

N = 100000
M = 9
NC = 2
NS = 16
NW = NC * NS
BLK = 160
NBLK = 20
IDXB = BLK * M
CPT = BLK * NBLK
NP = NW * CPT
B0 = 20
B1 = 2 * NBLK - B0

_params = pltpu.CompilerParams(
    needs_layout_passes=False, use_tc_tiling_on_sc=False
)


def _make_mesh():
    return plsc.VectorSubcoreMesh(
        core_axis_name="c", subcore_axis_name="s", num_cores=NC, num_subcores=NS
    )


def _wid():
    return lax.axis_index("s") * NC + lax.axis_index("c")


def _full16(v):
    return jnp.full((16,), v, jnp.int32)


def _phase0(xt, utf, u1f, invt, out_t1, xv, uv, u1v, iv, outb, sem):
    w = _wid()
    iota = lax.iota(jnp.int32, 16)

    def block(b, carry):
        base = w * CPT + b * BLK
        pltpu.async_copy(xt.at[:, pl.ds(base, BLK)], xv, sem)
        pltpu.async_copy(utf.at[pl.ds(base, BLK)], uv, sem)
        pltpu.async_copy(u1f.at[pl.ds(base, BLK)], u1v, sem)
        pltpu.async_copy(invt.at[:, pl.ds(base, BLK)], iv, sem)
        pltpu.make_async_copy(xt.at[:, pl.ds(0, BLK)], xv, sem).wait()
        pltpu.make_async_copy(utf.at[pl.ds(0, BLK)], uv, sem).wait()
        pltpu.make_async_copy(u1f.at[pl.ds(0, BLK)], u1v, sem).wait()
        pltpu.make_async_copy(invt.at[:, pl.ds(0, BLK)], iv, sem).wait()
        for p in range(0, BLK, 16):
            rows = iota + p
            plsc.store_scatter(outb, [rows, _full16(0)], xv[0, pl.ds(p, 16)])
            plsc.store_scatter(outb, [rows, _full16(1)], xv[1, pl.ds(p, 16)])
            plsc.store_scatter(outb, [rows, _full16(2)], uv[pl.ds(p, 16)])
            plsc.store_scatter(outb, [rows, _full16(3)], u1v[pl.ds(p, 16)])
            plsc.store_scatter(outb, [rows, _full16(4)], iv[0, pl.ds(p, 16)])
            plsc.store_scatter(outb, [rows, _full16(5)], iv[1, pl.ds(p, 16)])
            plsc.store_scatter(outb, [rows, _full16(6)], iv[2, pl.ds(p, 16)])
            plsc.store_scatter(outb, [rows, _full16(7)], iv[3, pl.ds(p, 16)])
        pltpu.sync_copy(outb, out_t1.at[pl.ds(base, BLK)])
        return carry

    lax.fori_loop(0, NBLK, block, 0)


def _phase1(table1, nidx, upf, usf, out_t2, out_gp, out_part,
            idx_a, idx_b, buf_a, buf_b, own_a, own_b, up_a, up_b, us_a, us_b,
            outb, outp, resv, tabs,
            sia, sib, sga, sgb, soa, sob, spa, spb, sqa, sqb):
    w = _wid()
    c = lax.axis_index("c")
    sidx = lax.axis_index("s")
    myblk = jnp.where(c == 0, B0, B1)
    gbase = jnp.where(c == 0, sidx * B0, NS * B0 + sidx * B1)
    iota = lax.iota(jnp.int32, 16)
    sets = [(idx_a, buf_a, own_a, up_a, us_a, sia, sga, soa, spa, sqa),
            (idx_b, buf_b, own_b, up_b, us_b, sib, sgb, sob, spb, sqb)]
    slc = NP // NS
    pltpu.sync_copy(table1.at[pl.ds(sidx * slc, slc)],
                    tabs.at[pl.ds(sidx * slc, slc)])
    plsc.subcore_barrier()

    def issue_idx(b, s):
        idx_v, buf, ownv, upv, usv, si, sg, so, sp, sq = s
        base = (gbase + b) * BLK
        for m in range(M):
            pltpu.async_copy(nidx.at[m, pl.ds(base, BLK)],
                             idx_v.at[pl.ds(m * BLK, BLK)], si)

    def wait_idx_fire(b, s):
        idx_v, buf, ownv, upv, usv, si, sg, so, sp, sq = s
        base = (gbase + b) * BLK
        for m in range(M):
            pltpu.make_async_copy(nidx.at[m, pl.ds(0, BLK)],
                                  idx_v.at[pl.ds(m * BLK, BLK)], si).wait()
        pltpu.async_copy(tabs.at[idx_v], buf, sg)
        pltpu.async_copy(table1.at[pl.ds(base, BLK)], ownv, so)
        pltpu.async_copy(upf.at[pl.ds(base, BLK)], upv, sp)
        pltpu.async_copy(usf.at[pl.ds(base, BLK)], usv, sq)

    def wait(s):
        idx_v, buf, ownv, upv, usv, si, sg, so, sp, sq = s
        pltpu.make_async_copy(tabs.at[idx_v], buf, sg).wait()
        pltpu.make_async_copy(table1.at[pl.ds(0, BLK)], ownv, so).wait()
        pltpu.make_async_copy(upf.at[pl.ds(0, BLK)], upv, sp).wait()
        pltpu.make_async_copy(usf.at[pl.ds(0, BLK)], usv, sq).wait()

    def compute(b, s, accu):
        idx_v, buf, ownv, upv, usv, si, sg, so, sp, sq = s
        base = (gbase + b) * BLK
        for p in range(0, BLK, 16):
            rows = iota + p
            xi0 = plsc.load_gather(ownv, [rows, _full16(0)])
            xi1 = plsc.load_gather(ownv, [rows, _full16(1)])
            uti = plsc.load_gather(ownv, [rows, _full16(2)])
            u1i = plsc.load_gather(ownv, [rows, _full16(3)])
            s00 = jnp.zeros((16,), jnp.float32)
            s01 = jnp.zeros((16,), jnp.float32)
            s10 = jnp.zeros((16,), jnp.float32)
            s11 = jnp.zeros((16,), jnp.float32)
            for m in range(M):
                rm = rows + m * BLK
                xj0 = plsc.load_gather(buf, [rm, _full16(0)])
                xj1 = plsc.load_gather(buf, [rm, _full16(1)])
                uj = plsc.load_gather(buf, [rm, _full16(2)])
                u1j = plsc.load_gather(buf, [rm, _full16(3)])
                xd0 = xj0 - xi0
                xd1 = xj1 - xi1
                ud = uj - uti
                ud1 = u1j - u1i
                s00 = s00 + ud * xd0
                s01 = s01 + ud * xd1
                s10 = s10 + ud1 * xd0
                s11 = s11 + ud1 * xd1
            ia = plsc.load_gather(ownv, [rows, _full16(4)])
            ib = plsc.load_gather(ownv, [rows, _full16(5)])
            ic = plsc.load_gather(ownv, [rows, _full16(6)])
            id_ = plsc.load_gather(ownv, [rows, _full16(7)])
            gx = s00 * ia + s01 * ic
            gy = s00 * ib + s01 * id_
            gx1 = s10 * ia + s11 * ic
            gy1 = s10 * ib + s11 * id_
            plsc.store_scatter(outb, [rows, _full16(0)], xi0)
            plsc.store_scatter(outb, [rows, _full16(1)], xi1)
            plsc.store_scatter(outb, [rows, _full16(2)], gx)
            plsc.store_scatter(outb, [rows, _full16(3)], gy)
            plsc.store_scatter(outb, [rows, _full16(4)], gx1)
            plsc.store_scatter(outb, [rows, _full16(5)], gy1)
            outp[0, pl.ds(p, 16)] = gx
            outp[1, pl.ds(p, 16)] = gy
            outp[2, pl.ds(p, 16)] = gx1
            outp[3, pl.ds(p, 16)] = gy1
            du = upv[pl.ds(p, 16)] - usv[pl.ds(p, 16)]
            accu = accu + du * du
        pltpu.sync_copy(outb, out_t2.at[pl.ds(base, BLK)])
        pltpu.sync_copy(outp, out_gp.at[:, pl.ds(base, BLK)])
        return accu

    @pl.when(myblk >= 2)
    def _():
        issue_idx(0, sets[0])
        issue_idx(1, sets[1])
        wait_idx_fire(0, sets[0])

    zero = jnp.zeros((16,), jnp.float32)

    def body(g, accu):
        b0 = 2 * g
        for sub in (0, 1):
            b = b0 + sub
            sq_, snx = sets[sub], sets[1 - sub]

            @pl.when(b + 1 < myblk)
            def _():
                wait_idx_fire(b + 1, snx)

            wait(sq_)

            @pl.when(b + 2 < myblk)
            def _():
                issue_idx(b + 2, sq_)

            accu = compute(b, sq_, accu)
        return accu

    accu = lax.fori_loop(0, myblk // 2, body, zero)
    resv[...] = accu
    pltpu.sync_copy(resv, out_part.at[w])


def _phase2(table2, table1, gp, nidx, out_part,
            idx_a, idx_b, buf_a, buf_b, own_a, own_b, gp_a, gp_b, resv, tabs,
            sia, sib, sga, sgb, soa, sob, spa, spb):
    w = _wid()
    c = lax.axis_index("c")
    sidx = lax.axis_index("s")
    myblk = jnp.where(c == 0, B0, B1)
    gbase = jnp.where(c == 0, sidx * B0, NS * B0 + sidx * B1)
    iota = lax.iota(jnp.int32, 16)
    sets = [(idx_a, buf_a, own_a, gp_a, sia, sga, soa, spa),
            (idx_b, buf_b, own_b, gp_b, sib, sgb, sob, spb)]
    slc = NP // NS
    pltpu.sync_copy(table2.at[pl.ds(sidx * slc, slc)],
                    tabs.at[pl.ds(sidx * slc, slc)])
    plsc.subcore_barrier()

    def issue_idx(b, s):
        idx_v, buf, ownv, gpv, si, sg, so, sp = s
        base = (gbase + b) * BLK
        for m in range(M):
            pltpu.async_copy(nidx.at[m, pl.ds(base, BLK)],
                             idx_v.at[pl.ds(m * BLK, BLK)], si)

    def wait_idx_fire(b, s):
        idx_v, buf, ownv, gpv, si, sg, so, sp = s
        base = (gbase + b) * BLK
        for m in range(M):
            pltpu.make_async_copy(nidx.at[m, pl.ds(0, BLK)],
                                  idx_v.at[pl.ds(m * BLK, BLK)], si).wait()
        pltpu.async_copy(tabs.at[idx_v], buf, sg)
        pltpu.async_copy(table1.at[pl.ds(base, BLK)], ownv, so)
        pltpu.async_copy(gp.at[:, pl.ds(base, BLK)], gpv, sp)

    def wait(s):
        idx_v, buf, ownv, gpv, si, sg, so, sp = s
        pltpu.make_async_copy(tabs.at[idx_v], buf, sg).wait()
        pltpu.make_async_copy(table1.at[pl.ds(0, BLK)], ownv, so).wait()
        pltpu.make_async_copy(gp.at[:, pl.ds(0, BLK)], gpv, sp).wait()

    def compute(b, s, accf):
        idx_v, buf, ownv, gpv, si, sg, so, sp = s
        base = (gbase + b) * BLK
        for p in range(0, BLK, 16):
            rows = iota + p
            xi0 = plsc.load_gather(ownv, [rows, _full16(0)])
            xi1 = plsc.load_gather(ownv, [rows, _full16(1)])
            uti = plsc.load_gather(ownv, [rows, _full16(2)])
            u1i = plsc.load_gather(ownv, [rows, _full16(3)])
            gi0 = gpv[0, pl.ds(p, 16)]
            gi1 = gpv[1, pl.ds(p, 16)]
            gi2 = gpv[2, pl.ds(p, 16)]
            gi3 = gpv[3, pl.ds(p, 16)]
            a00 = jnp.zeros((16,), jnp.float32)
            a01 = jnp.zeros((16,), jnp.float32)
            a10 = jnp.zeros((16,), jnp.float32)
            a11 = jnp.zeros((16,), jnp.float32)
            b00 = jnp.zeros((16,), jnp.float32)
            b01 = jnp.zeros((16,), jnp.float32)
            b10 = jnp.zeros((16,), jnp.float32)
            b11 = jnp.zeros((16,), jnp.float32)
            for m in range(M):
                rm = rows + m * BLK
                xj0 = plsc.load_gather(buf, [rm, _full16(0)])
                xj1 = plsc.load_gather(buf, [rm, _full16(1)])
                gj0 = plsc.load_gather(buf, [rm, _full16(2)])
                gj1 = plsc.load_gather(buf, [rm, _full16(3)])
                gj2 = plsc.load_gather(buf, [rm, _full16(4)])
                gj3 = plsc.load_gather(buf, [rm, _full16(5)])
                xd0 = xj0 - xi0
                xd1 = xj1 - xi1
                gd0 = gj0 - gi0
                gd1 = gj1 - gi1
                gd2 = gj2 - gi2
                gd3 = gj3 - gi3
                a00 = a00 + gd0 * xd0
                a01 = a01 + gd0 * xd1
                a10 = a10 + gd1 * xd0
                a11 = a11 + gd1 * xd1
                b00 = b00 + gd2 * xd0
                b01 = b01 + gd2 * xd1
                b10 = b10 + gd3 * xd0
                b11 = b11 + gd3 * xd1
            ia = plsc.load_gather(ownv, [rows, _full16(4)])
            ib = plsc.load_gather(ownv, [rows, _full16(5)])
            ic = plsc.load_gather(ownv, [rows, _full16(6)])
            id_ = plsc.load_gather(ownv, [rows, _full16(7)])
            lap = a00 * ia + a01 * ic + a10 * ib + a11 * id_
            lap1 = b00 * ia + b01 * ic + b10 * ib + b11 * id_
            f = u1i - uti - 0.01 * (
                0.01 * lap + uti - uti * uti * uti
                + 0.01 * lap1 + u1i - u1i * u1i * u1i
            )
            gid = base + p + iota
            f = jnp.where(gid < N, f, 0.0)
            accf = accf + f * f
        return accf

    @pl.when(myblk >= 2)
    def _():
        issue_idx(0, sets[0])
        issue_idx(1, sets[1])
        wait_idx_fire(0, sets[0])

    zero = jnp.zeros((16,), jnp.float32)

    def body(g, accf):
        b0 = 2 * g
        for sub in (0, 1):
            b = b0 + sub
            sq_, snx = sets[sub], sets[1 - sub]

            @pl.when(b + 1 < myblk)
            def _():
                wait_idx_fire(b + 1, snx)

            wait(sq_)

            @pl.when(b + 2 < myblk)
            def _():
                issue_idx(b + 2, sq_)

            accf = compute(b, sq_, accf)
        return accf

    accf = lax.fori_loop(0, myblk // 2, body, zero)
    resv[...] = accf
    pltpu.sync_copy(resv, out_part.at[w])


def kernel(up, usol, ut, x_to_train_f, ut1, n_index, inv_mat):
    pad = NP - N
    xt = jnp.pad(x_to_train_f.T, ((0, 0), (0, pad)))
    utf = jnp.pad(ut.reshape(-1), (0, pad))
    u1f = jnp.pad(ut1.reshape(-1), (0, pad))
    invt = jnp.pad(inv_mat.transpose(1, 2, 0).reshape(4, N),
                   ((0, 0), (0, pad)))
    nidx = jnp.pad(n_index.astype(jnp.int32).T, ((0, 0), (0, pad)))
    upf = jnp.pad(up.reshape(-1), (0, pad))
    usf = jnp.pad(usol.reshape(-1), (0, pad))

    p0 = pl.kernel(
        _phase0,
        out_type=jax.ShapeDtypeStruct((NP, 8), jnp.float32),
        mesh=_make_mesh(),
        compiler_params=_params,
        scratch_types=[
            pltpu.VMEM((2, BLK), jnp.float32),
            pltpu.VMEM((BLK,), jnp.float32),
            pltpu.VMEM((BLK,), jnp.float32),
            pltpu.VMEM((4, BLK), jnp.float32),
            pltpu.VMEM((BLK, 8), jnp.float32),
            pltpu.SemaphoreType.DMA,
        ],
    )
    table1 = p0(xt, utf, u1f, invt)

    p1 = pl.kernel(
        _phase1,
        out_type=(
            jax.ShapeDtypeStruct((NP, 8), jnp.float32),
            jax.ShapeDtypeStruct((4, NP), jnp.float32),
            jax.ShapeDtypeStruct((NW, 16), jnp.float32),
        ),
        mesh=_make_mesh(),
        compiler_params=_params,
        scratch_types=[
            pltpu.VMEM((IDXB,), jnp.int32),
            pltpu.VMEM((IDXB,), jnp.int32),
            pltpu.VMEM((IDXB, 8), jnp.float32),
            pltpu.VMEM((IDXB, 8), jnp.float32),
            pltpu.VMEM((BLK, 8), jnp.float32),
            pltpu.VMEM((BLK, 8), jnp.float32),
            pltpu.VMEM((BLK,), jnp.float32),
            pltpu.VMEM((BLK,), jnp.float32),
            pltpu.VMEM((BLK,), jnp.float32),
            pltpu.VMEM((BLK,), jnp.float32),
            pltpu.VMEM((BLK, 8), jnp.float32),
            pltpu.VMEM((4, BLK), jnp.float32),
            pltpu.VMEM((16,), jnp.float32),
            pltpu.VMEM_SHARED((NP, 8), jnp.float32),
            pltpu.SemaphoreType.DMA,
            pltpu.SemaphoreType.DMA,
            pltpu.SemaphoreType.DMA,
            pltpu.SemaphoreType.DMA,
            pltpu.SemaphoreType.DMA,
            pltpu.SemaphoreType.DMA,
            pltpu.SemaphoreType.DMA,
            pltpu.SemaphoreType.DMA,
            pltpu.SemaphoreType.DMA,
            pltpu.SemaphoreType.DMA,
        ],
    )
    table2, gp, part1 = p1(table1, nidx, upf, usf)

    p2 = pl.kernel(
        _phase2,
        out_type=jax.ShapeDtypeStruct((NW, 16), jnp.float32),
        mesh=_make_mesh(),
        compiler_params=_params,
        scratch_types=[
            pltpu.VMEM((IDXB,), jnp.int32),
            pltpu.VMEM((IDXB,), jnp.int32),
            pltpu.VMEM((IDXB, 8), jnp.float32),
            pltpu.VMEM((IDXB, 8), jnp.float32),
            pltpu.VMEM((BLK, 8), jnp.float32),
            pltpu.VMEM((BLK, 8), jnp.float32),
            pltpu.VMEM((4, BLK), jnp.float32),
            pltpu.VMEM((4, BLK), jnp.float32),
            pltpu.VMEM((16,), jnp.float32),
            pltpu.VMEM_SHARED((NP, 8), jnp.float32),
            pltpu.SemaphoreType.DMA,
            pltpu.SemaphoreType.DMA,
            pltpu.SemaphoreType.DMA,
            pltpu.SemaphoreType.DMA,
            pltpu.SemaphoreType.DMA,
            pltpu.SemaphoreType.DMA,
            pltpu.SemaphoreType.DMA,
            pltpu.SemaphoreType.DMA,
        ],
    )
    part2 = p2(table2, table1, gp, nidx)
    return jnp.sum(part1) + 4.0 * jnp.sum(part2)
